# baseline (speedup 1.0000x reference)
import functools
import math

import jax
import jax.numpy as jnp
from jax import lax
from jax.experimental import pallas as pl
from jax.experimental.pallas import tpu as pltpu
from jax.experimental.pallas import tpu_sc as plsc

_N = 2_000_000
_G = 1000
_LW = 1.0
_SIZE = _LW / _G
_DZ = _SIZE / 2.0
_A = 0.0136

_C = 8000
_NCHUNK = _N // _C
_INFO = plsc.get_sparse_core_info()
_NC = _INFO.num_cores
_NS = _INFO.num_subcores
_NW = _NC * _NS
_L = 16

_TB = 131072
_TGRID = (_N + _TB - 1) // _TB

_SENT = _G * _G

_mesh = plsc.VectorSubcoreMesh(core_axis_name="c", subcore_axis_name="s")


@functools.partial(
    pl.kernel, mesh=_mesh,
    out_type=jax.ShapeDtypeStruct((_N,), jnp.float32),
    scratch_types=[
        pltpu.VMEM((_C,), jnp.float32),
        pltpu.VMEM((_C,), jnp.float32),
        pltpu.VMEM((_C,), jnp.int32),
        pltpu.VMEM((_C,), jnp.float32),
        pltpu.SemaphoreType.DMA,
    ],
)
def _sc_rl_gather(x_hbm, y_hbm, tab_hbm, out_hbm, xb, yb, idxb, gatb, sem):
  wid = lax.axis_index("s") * _NC + lax.axis_index("c")

  def chunk(i, carry):
    base = (wid + i * _NW) * _C
    pltpu.sync_copy(x_hbm.at[pl.ds(base, _C)], xb)
    pltpu.sync_copy(y_hbm.at[pl.ds(base, _C)], yb)

    def step(j, c2):
      xv = xb[pl.ds(j * _L, _L)]
      yv = yb[pl.ds(j * _L, _L)]
      ix = jnp.minimum(jnp.maximum((xv / _SIZE).astype(jnp.int32), 0), _G - 1)
      iy = jnp.minimum(jnp.maximum((yv / _SIZE).astype(jnp.int32), 0), _G - 1)
      idxb[pl.ds(j * _L, _L)] = ix * _G + iy
      return c2

    lax.fori_loop(0, _C // _L, step, 0)
    pltpu.async_copy(tab_hbm.at[idxb], gatb, sem).wait()
    pltpu.sync_copy(gatb, out_hbm.at[pl.ds(base, _C)])
    return carry

  n_w = (_NCHUNK - wid + _NW - 1) // _NW
  lax.fori_loop(0, n_w, chunk, 0)


@functools.partial(
    pl.kernel, mesh=_mesh,
    out_type=(jax.ShapeDtypeStruct((_N,), jnp.float32),
              jax.ShapeDtypeStruct((_N,), jnp.float32)),
    scratch_types=[
        pltpu.VMEM((_C,), jnp.int32),
        pltpu.VMEM((_C,), jnp.float32),
        pltpu.VMEM((_C,), jnp.float32),
        pltpu.VMEM((_C,), jnp.float32),
        pltpu.VMEM((_C,), jnp.float32),
        pltpu.VMEM((_C,), jnp.float32),
        pltpu.VMEM((2 * _C,), jnp.float32),
        pltpu.SemaphoreType.DMA,
    ],
)
def _sc_res_hits(f2_hbm, xp_hbm, yp_hbm, nx_hbm, ny_hbm, tab_hbm,
                 hx_hbm, hy_hbm, f2b, xpb, ypb, nxb, nyb, resb, outb, sem):
  wid = lax.axis_index("s") * _NC + lax.axis_index("c")

  def chunk(i, carry):
    base = (wid + i * _NW) * _C
    pltpu.sync_copy(f2_hbm.at[pl.ds(base, _C)], f2b)
    pltpu.sync_copy(xp_hbm.at[pl.ds(base, _C)], xpb)
    pltpu.sync_copy(yp_hbm.at[pl.ds(base, _C)], ypb)
    pltpu.sync_copy(nx_hbm.at[pl.ds(base, _C)], nxb)
    pltpu.sync_copy(ny_hbm.at[pl.ds(base, _C)], nyb)
    pltpu.async_copy(tab_hbm.at[f2b], resb, sem).wait()

    def step(j, c2):
      sl = pl.ds(j * _L, _L)
      d = jnp.abs(resb[sl]) + 1e-17
      hx = xpb[sl] + nxb[sl] / d
      hy = ypb[sl] + nyb[sl] / d
      outb[sl] = hx
      outb[pl.ds(_C + j * _L, _L)] = hy
      return c2

    lax.fori_loop(0, _C // _L, step, 0)
    pltpu.sync_copy(outb.at[pl.ds(0, _C)], hx_hbm.at[pl.ds(base, _C)])
    pltpu.sync_copy(outb.at[pl.ds(_C, _C)], hy_hbm.at[pl.ds(base, _C)])
    return carry

  n_w = (_NCHUNK - wid + _NW - 1) // _NW
  lax.fori_loop(0, n_w, chunk, 0)


def _tc_math_body(x_ref, y_ref, th_ref, tx_ref, ty_ref, p_ref, z1_ref, z2_ref,
                  u_ref, rl_ref, xp_ref, yp_ref, f2_ref):
  x = x_ref[...]
  y = y_ref[...]
  theta = th_ref[...]
  theta_x = tx_ref[...]
  theta_y = ty_ref[...]
  p = p_ref[...]
  z1 = z1_ref[...]
  z2 = z2_ref[...]
  u = u_ref[...]
  rl = rl_ref[...]

  mask = (x >= 0.0) & (x < _LW) & (y >= 0.0) & (y < _LW)
  x0 = _DZ / (rl * jnp.cos(theta))
  theta0 = _A / p * jnp.sqrt(x0)
  phi = u * 2.0 * math.pi
  dh = _DZ * jnp.sin(theta0) * (z1 / math.sqrt(12.0) + z2 / 2.0)
  dx = math.sqrt(2.0) * dh * jnp.cos(phi) * jnp.cos(theta_x)
  dy = math.sqrt(2.0) * dh * jnp.sin(phi) * jnp.cos(theta_y)
  xn = jnp.where(mask, x + dx, x)
  yn = jnp.where(mask, y + dy, y)
  xn = xn + _DZ * jnp.tan(theta_x)
  yn = yn + _DZ * jnp.tan(theta_y)

  mask1 = (xn >= 0.0) & (xn < _LW) & (yn >= 0.0) & (yn < _LW)
  ix = jnp.clip(jnp.floor(xn / _SIZE).astype(jnp.int32), 0, _G - 1)
  iy = jnp.clip(jnp.floor(yn / _SIZE).astype(jnp.int32), 0, _G - 1)
  f2 = jnp.where(mask1, ix * _G + iy, _SENT)

  xp_ref[...] = xn
  yp_ref[...] = yn
  f2_ref[...] = f2


_B1 = pl.BlockSpec((_TB,), lambda i: (i,))

_tc_math = pl.pallas_call(
    _tc_math_body,
    grid=(_TGRID,),
    in_specs=[_B1] * 10,
    out_specs=[_B1] * 3,
    out_shape=[
        jax.ShapeDtypeStruct((_N,), jnp.float32),
        jax.ShapeDtypeStruct((_N,), jnp.float32),
        jax.ShapeDtypeStruct((_N,), jnp.int32),
    ],
)


def kernel(x, y, theta, theta_x, theta_y, p, z1a, z2a, ua, z1b, z2b, ub,
           nx, ny, resolution, efficiency, rad_length):
  tab1 = rad_length.reshape(-1)
  tab2 = jnp.concatenate(
      [resolution.reshape(-1), jnp.zeros((8,), jnp.float32)])

  rl = _sc_rl_gather(x, y, tab1)
  xp, yp, f2 = _tc_math(x, y, theta, theta_x, theta_y, p, z1a, z2a, ua, rl)
  hx, hy = _sc_res_hits(f2, xp, yp, nx, ny, tab2)
  return jnp.stack([hx, hy], axis=1)


# speedup vs baseline: 1.0814x; 1.0814x over previous
import functools
import math

import jax
import jax.numpy as jnp
from jax import lax
from jax.experimental import pallas as pl
from jax.experimental.pallas import tpu as pltpu
from jax.experimental.pallas import tpu_sc as plsc

_N = 2_000_000
_G = 1000
_LW = 1.0
_SIZE = _LW / _G
_DZ = _SIZE / 2.0
_A = 0.0136

_INFO = plsc.get_sparse_core_info()
_NC = _INFO.num_cores
_NS = _INFO.num_subcores
_NW = _NC * _NS
_L = 16

_C1 = 8000
_NCHUNK1 = _N // _C1
_K1 = (_NCHUNK1 + _NW - 1) // _NW

_C2 = 4000
_NCHUNK2 = _N // _C2
_K2 = (_NCHUNK2 + _NW - 1) // _NW

_TB = 131072
_TGRID = (_N + _TB - 1) // _TB

_SENT = _G * _G

_mesh = plsc.VectorSubcoreMesh(core_axis_name="c", subcore_axis_name="s")


@functools.partial(
    pl.kernel, mesh=_mesh,
    out_type=jax.ShapeDtypeStruct((_N,), jnp.float32),
    scratch_types=(
        [pltpu.VMEM((_C1,), jnp.float32) for _ in range(2)]
        + [pltpu.VMEM((_C1,), jnp.float32) for _ in range(2)]
        + [pltpu.VMEM((_C1,), jnp.int32) for _ in range(2)]
        + [pltpu.VMEM((_C1,), jnp.float32) for _ in range(2)]
        + [pltpu.SemaphoreType.DMA for _ in range(5)]
    ),
)
def _sc_rl_gather(x_hbm, y_hbm, tab_hbm, out_hbm,
                  xb0, xb1, yb0, yb1, ib0, ib1, gb0, gb1,
                  semL0, semL1, semG, semS0, semS1):
  wid = lax.axis_index("s") * _NC + lax.axis_index("c")
  xb = (xb0, xb1)
  yb = (yb0, yb1)
  ib = (ib0, ib1)
  gb = (gb0, gb1)
  semL = (semL0, semL1)
  semS = (semS0, semS1)

  def cbase(k):
    return (wid + k * _NW) * _C1

  def guard(k):
    return wid + k * _NW < _NCHUNK1

  @pl.when(guard(0))
  def _():
    pltpu.async_copy(x_hbm.at[pl.ds(cbase(0), _C1)], xb[0], semL[0])
    pltpu.async_copy(y_hbm.at[pl.ds(cbase(0), _C1)], yb[0], semL[0])

  for k in range(_K1):
    b = k % 2
    nb = (k + 1) % 2

    @pl.when(guard(k))
    def _(k=k, b=b):
      pltpu.make_async_copy(x_hbm.at[pl.ds(cbase(k), _C1)], xb[b],
                            semL[b]).wait()
      pltpu.make_async_copy(y_hbm.at[pl.ds(cbase(k), _C1)], yb[b],
                            semL[b]).wait()

    if k + 1 < _K1:
      @pl.when(guard(k + 1))
      def _(k=k, nb=nb):
        pltpu.async_copy(x_hbm.at[pl.ds(cbase(k + 1), _C1)], xb[nb], semL[nb])
        pltpu.async_copy(y_hbm.at[pl.ds(cbase(k + 1), _C1)], yb[nb], semL[nb])

    @pl.when(guard(k))
    def _(k=k, b=b):
      def step(j, c2):
        xv = xb[b][pl.ds(j * _L, _L)]
        yv = yb[b][pl.ds(j * _L, _L)]
        ix = jnp.minimum(jnp.maximum((xv / _SIZE).astype(jnp.int32), 0),
                         _G - 1)
        iy = jnp.minimum(jnp.maximum((yv / _SIZE).astype(jnp.int32), 0),
                         _G - 1)
        ib[b][pl.ds(j * _L, _L)] = ix * _G + iy
        return c2

      lax.fori_loop(0, _C1 // _L, step, 0)

    if k >= 2:
      @pl.when(guard(k - 2))
      def _(k=k, b=b):
        pltpu.make_async_copy(gb[b], out_hbm.at[pl.ds(cbase(k - 2), _C1)],
                              semS[b]).wait()

    @pl.when(guard(k))
    def _(k=k, b=b):
      pltpu.async_copy(tab_hbm.at[ib[b]], gb[b], semG).wait()
      pltpu.async_copy(gb[b], out_hbm.at[pl.ds(cbase(k), _C1)], semS[b])

  for k in (_K1 - 2, _K1 - 1):
    b = k % 2

    @pl.when(guard(k))
    def _(k=k, b=b):
      pltpu.make_async_copy(gb[b], out_hbm.at[pl.ds(cbase(k), _C1)],
                            semS[b]).wait()


@functools.partial(
    pl.kernel, mesh=_mesh,
    out_type=(jax.ShapeDtypeStruct((_N,), jnp.float32),
              jax.ShapeDtypeStruct((_N,), jnp.float32)),
    scratch_types=(
        [pltpu.VMEM((_C2,), jnp.int32) for _ in range(2)]
        + [pltpu.VMEM((_C2,), jnp.float32) for _ in range(2)]
        + [pltpu.VMEM((_C2,), jnp.float32) for _ in range(2)]
        + [pltpu.VMEM((_C2,), jnp.float32) for _ in range(2)]
        + [pltpu.VMEM((_C2,), jnp.float32) for _ in range(2)]
        + [pltpu.VMEM((_C2,), jnp.float32) for _ in range(2)]
        + [pltpu.VMEM((_C2,), jnp.float32) for _ in range(2)]
        + [pltpu.VMEM((_C2,), jnp.float32) for _ in range(2)]
        + [pltpu.SemaphoreType.DMA for _ in range(5)]
    ),
)
def _sc_res_hits(f2_hbm, xp_hbm, yp_hbm, nx_hbm, ny_hbm, tab_hbm,
                 hx_hbm, hy_hbm,
                 f2b0, f2b1, xpb0, xpb1, ypb0, ypb1, nxb0, nxb1,
                 nyb0, nyb1, resb0, resb1, hxb0, hxb1, hyb0, hyb1,
                 semL0, semL1, semG, semS0, semS1):
  wid = lax.axis_index("s") * _NC + lax.axis_index("c")
  f2b = (f2b0, f2b1)
  xpb = (xpb0, xpb1)
  ypb = (ypb0, ypb1)
  nxb = (nxb0, nxb1)
  nyb = (nyb0, nyb1)
  resb = (resb0, resb1)
  hxb = (hxb0, hxb1)
  hyb = (hyb0, hyb1)
  semL = (semL0, semL1)
  semS = (semS0, semS1)
  ins = (f2_hbm, xp_hbm, yp_hbm, nx_hbm, ny_hbm)

  def bufs(b):
    return (f2b[b], xpb[b], ypb[b], nxb[b], nyb[b])

  def cbase(k):
    return (wid + k * _NW) * _C2

  def guard(k):
    return wid + k * _NW < _NCHUNK2

  def issue_loads(k, b):
    base = pl.ds(cbase(k), _C2)
    for src, dst in zip(ins, bufs(b)):
      pltpu.async_copy(src.at[base], dst, semL[b])

  def wait_loads(k, b):
    base = pl.ds(cbase(k), _C2)
    for src, dst in zip(ins, bufs(b)):
      pltpu.make_async_copy(src.at[base], dst, semL[b]).wait()

  @pl.when(guard(0))
  def _():
    issue_loads(0, 0)

  for k in range(_K2):
    b = k % 2
    nb = (k + 1) % 2

    @pl.when(guard(k))
    def _(k=k, b=b):
      wait_loads(k, b)

    if k + 1 < _K2:
      @pl.when(guard(k + 1))
      def _(k=k, nb=nb):
        issue_loads(k + 1, nb)

    if k >= 2:
      @pl.when(guard(k - 2))
      def _(k=k, b=b):
        pltpu.make_async_copy(hxb[b], hx_hbm.at[pl.ds(cbase(k - 2), _C2)],
                              semS[b]).wait()
        pltpu.make_async_copy(hyb[b], hy_hbm.at[pl.ds(cbase(k - 2), _C2)],
                              semS[b]).wait()

    @pl.when(guard(k))
    def _(k=k, b=b):
      pltpu.async_copy(tab_hbm.at[f2b[b]], resb[b], semG).wait()

      def step(j, c2):
        sl = pl.ds(j * _L, _L)
        d = jnp.abs(resb[b][sl]) + 1e-17
        hxb[b][sl] = xpb[b][sl] + nxb[b][sl] / d
        hyb[b][sl] = ypb[b][sl] + nyb[b][sl] / d
        return c2

      lax.fori_loop(0, _C2 // _L, step, 0)
      pltpu.async_copy(hxb[b], hx_hbm.at[pl.ds(cbase(k), _C2)], semS[b])
      pltpu.async_copy(hyb[b], hy_hbm.at[pl.ds(cbase(k), _C2)], semS[b])

  for k in (_K2 - 2, _K2 - 1):
    b = k % 2

    @pl.when(guard(k))
    def _(k=k, b=b):
      pltpu.make_async_copy(hxb[b], hx_hbm.at[pl.ds(cbase(k), _C2)],
                            semS[b]).wait()
      pltpu.make_async_copy(hyb[b], hy_hbm.at[pl.ds(cbase(k), _C2)],
                            semS[b]).wait()


def _tc_math_body(x_ref, y_ref, th_ref, tx_ref, ty_ref, p_ref, z1_ref, z2_ref,
                  u_ref, rl_ref, xp_ref, yp_ref, f2_ref):
  x = x_ref[...]
  y = y_ref[...]
  theta = th_ref[...]
  theta_x = tx_ref[...]
  theta_y = ty_ref[...]
  p = p_ref[...]
  z1 = z1_ref[...]
  z2 = z2_ref[...]
  u = u_ref[...]
  rl = rl_ref[...]

  mask = (x >= 0.0) & (x < _LW) & (y >= 0.0) & (y < _LW)
  x0 = _DZ / (rl * jnp.cos(theta))
  theta0 = _A / p * jnp.sqrt(x0)
  phi = u * 2.0 * math.pi
  dh = _DZ * jnp.sin(theta0) * (z1 / math.sqrt(12.0) + z2 / 2.0)
  dx = math.sqrt(2.0) * dh * jnp.cos(phi) * jnp.cos(theta_x)
  dy = math.sqrt(2.0) * dh * jnp.sin(phi) * jnp.cos(theta_y)
  xn = jnp.where(mask, x + dx, x)
  yn = jnp.where(mask, y + dy, y)
  xn = xn + _DZ * jnp.tan(theta_x)
  yn = yn + _DZ * jnp.tan(theta_y)

  mask1 = (xn >= 0.0) & (xn < _LW) & (yn >= 0.0) & (yn < _LW)
  ix = jnp.clip(jnp.floor(xn / _SIZE).astype(jnp.int32), 0, _G - 1)
  iy = jnp.clip(jnp.floor(yn / _SIZE).astype(jnp.int32), 0, _G - 1)
  f2 = jnp.where(mask1, ix * _G + iy, _SENT)

  xp_ref[...] = xn
  yp_ref[...] = yn
  f2_ref[...] = f2


_B1 = pl.BlockSpec((_TB,), lambda i: (i,))

_tc_math = pl.pallas_call(
    _tc_math_body,
    grid=(_TGRID,),
    in_specs=[_B1] * 10,
    out_specs=[_B1] * 3,
    out_shape=[
        jax.ShapeDtypeStruct((_N,), jnp.float32),
        jax.ShapeDtypeStruct((_N,), jnp.float32),
        jax.ShapeDtypeStruct((_N,), jnp.int32),
    ],
)


def kernel(x, y, theta, theta_x, theta_y, p, z1a, z2a, ua, z1b, z2b, ub,
           nx, ny, resolution, efficiency, rad_length):
  tab1 = rad_length.reshape(-1)
  tab2 = jnp.concatenate(
      [resolution.reshape(-1), jnp.zeros((8,), jnp.float32)])

  rl = _sc_rl_gather(x, y, tab1)
  xp, yp, f2 = _tc_math(x, y, theta, theta_x, theta_y, p, z1a, z2a, ua, rl)
  hx, hy = _sc_res_hits(f2, xp, yp, nx, ny, tab2)
  return jnp.stack([hx, hy], axis=1)
